# baseline (speedup 1.0000x reference)
import functools

import jax
import jax.numpy as jnp
from jax import lax
from jax.experimental import pallas as pl
from jax.experimental.pallas import tpu as pltpu
from jax.experimental.pallas import tpu_sc as plsc

N = 10000
E = 320000
D = 128
H = 16
C = 40

NC = 2
NS = 16
NW = NC * NS
CHUNK = 128
CPW = 80
IDX_BLK = 8
EPW = CPW * CHUNK
EPAD = NW * EPW
NP = 10112
RPT = NP // NS

ROWS_BLK = 1000


_MESH = plsc.VectorSubcoreMesh(core_axis_name="c", subcore_axis_name="s")
_SC_PARAMS = pltpu.CompilerParams(use_tc_tiling_on_sc=False)
MEGA = IDX_BLK * CHUNK
HCH = CPW // 2


def _zero_rows(ref, nrows, w):
    def zr(i, carry):
        for k in range(w // 16):
            ref[i, pl.ds(k * 16, 16)] = jnp.zeros((16,), jnp.float32)
        return carry
    lax.fori_loop(0, nrows, zr, 0)


def _fill_ones(ref, nrows, w):
    def orow(i, carry):
        for k in range(w // 16):
            ref[i, pl.ds(k * 16, 16)] = jnp.ones((16,), jnp.float32)
        return carry
    lax.fori_loop(0, nrows, orow, 0)


CH128_0 = 160
NM16_0 = 20
NMD_0 = 20
SPC = 32
NCH = EPAD // CHUNK
NMM = EPAD // MEGA


def _make_sc_pass128():
    scratch = [
        pltpu.VMEM((SPC, CHUNK), jnp.int32),
        pltpu.VMEM((SPC, CHUNK), jnp.int32),
        pltpu.VMEM((2, CHUNK, D), jnp.float32),
        pltpu.VMEM_SHARED((NP, D), jnp.float32),
        pltpu.SemaphoreType.DMA,
        pltpu.SemaphoreType.DMA,
        pltpu.SemaphoreType.DMA,
        pltpu.SemaphoreType.DMA,
    ]

    @functools.partial(
        pl.kernel, mesh=_MESH,
        out_type=(jax.ShapeDtypeStruct((NP, D), jnp.float32),),
        scratch_types=scratch, compiler_params=_SC_PARAMS)
    def sc_pass(feat, rows, cols, acc_out, row_v, col_v, g, acc_sh,
                gsem0, gsem1, ssem0, ssem1):
        c = lax.axis_index("c")
        s = lax.axis_index("s")
        base = s * RPT
        gsem = (gsem0, gsem1)
        ssem = (ssem0, ssem1)

        def fire_g(t, p):
            pltpu.async_copy(feat.at[col_v.at[t]], g.at[p], gsem[p])

        def wait_g(p):
            pltpu.make_async_copy(feat.at[col_v.at[0]], g.at[p], gsem[p]).wait()

        def fire_s(t, p):
            pltpu.async_copy(g.at[p], acc_sh.at[row_v.at[t]], ssem[p], add=True)

        def wait_s(p):
            pltpu.make_async_copy(g.at[p], acc_sh.at[row_v.at[0]], ssem[p]).wait()

        rem = RPT % CHUNK

        @pl.when(c == 0)
        def _():
            _zero_rows(g.at[0], CHUNK, D)
            for b in range(RPT // CHUNK):
                pltpu.sync_copy(g.at[0],
                                acc_sh.at[pl.ds(base + b * CHUNK, CHUNK)])
            if rem:
                pltpu.sync_copy(
                    g.at[0, pl.ds(0, rem)],
                    acc_sh.at[pl.ds(base + (RPT // CHUNK) * CHUNK, rem)])
        plsc.subcore_barrier()

        def run(chunk0, nstage):
            for stage in range(nstage):
                st = chunk0 + stage * SPC
                pltpu.sync_copy(rows.at[pl.ds(st, SPC)], row_v)
                pltpu.sync_copy(cols.at[pl.ds(st, SPC)], col_v)
                fire_g(0, 0)

                def body(jj2, carry):
                    tA = 2 * jj2
                    wait_g(0)

                    @pl.when(jj2 > 0)
                    def _():
                        wait_s(1)
                    fire_g(tA + 1, 1)
                    fire_s(tA, 0)
                    wait_g(1)
                    wait_s(0)

                    @pl.when(jj2 < SPC // 2 - 1)
                    def _():
                        fire_g(tA + 2, 0)
                    fire_s(tA + 1, 1)
                    return carry
                lax.fori_loop(0, SPC // 2, body, 0)
                wait_s(1)

        @pl.when(c == 0)
        def _():
            run(s * CH128_0, CH128_0 // SPC)

        plsc.subcore_barrier()

        @pl.when(c == 0)
        def _():
            for b in range(RPT // CHUNK):
                sl = pl.ds(base + b * CHUNK, CHUNK)
                pltpu.sync_copy(acc_sh.at[sl], acc_out.at[sl])
            if rem:
                sl = pl.ds(base + (RPT // CHUNK) * CHUNK, rem)
                pltpu.sync_copy(acc_sh.at[sl], acc_out.at[sl])

    return sc_pass


def _make_sc_pass16():
    scratch = [
        pltpu.VMEM((NM16_0, MEGA), jnp.int32),
        pltpu.VMEM((NM16_0, MEGA), jnp.int32),
        pltpu.VMEM((2, MEGA, H), jnp.float32),
        pltpu.VMEM_SHARED((NP, H), jnp.float32),
        pltpu.SemaphoreType.DMA,
        pltpu.SemaphoreType.DMA,
        pltpu.SemaphoreType.DMA,
        pltpu.SemaphoreType.DMA,
    ]

    @functools.partial(
        pl.kernel, mesh=_MESH,
        out_type=(jax.ShapeDtypeStruct((NP, H), jnp.float32),),
        scratch_types=scratch, compiler_params=_SC_PARAMS)
    def sc_pass(feat, rows, cols, acc_out, row_v, col_v, g, acc_sh,
                gsem0, gsem1, ssem0, ssem1):
        c = lax.axis_index("c")
        s = lax.axis_index("s")
        base = s * RPT
        gsem = (gsem0, gsem1)
        ssem = (ssem0, ssem1)

        def fire_g(m, p):
            pltpu.async_copy(feat.at[col_v.at[m]], g.at[p], gsem[p])

        def wait_g(p):
            pltpu.make_async_copy(feat.at[col_v.at[0]], g.at[p], gsem[p]).wait()

        def fire_s(m, p):
            pltpu.async_copy(g.at[p], acc_sh.at[row_v.at[m]], ssem[p], add=True)

        def wait_s(p):
            pltpu.make_async_copy(g.at[p], acc_sh.at[row_v.at[0]],
                                  ssem[p]).wait()

        @pl.when(c == 0)
        def _():
            _zero_rows(g.at[0], MEGA, H)
            pltpu.sync_copy(g.at[0, pl.ds(0, RPT)], acc_sh.at[pl.ds(base, RPT)])
        plsc.subcore_barrier()

        def run(m0, nm):
            pltpu.sync_copy(rows.at[pl.ds(m0, nm)], row_v.at[pl.ds(0, nm)])
            pltpu.sync_copy(cols.at[pl.ds(m0, nm)], col_v.at[pl.ds(0, nm)])
            fire_g(0, 0)
            for m in range(nm):
                p = m % 2
                wait_g(p)
                if m >= 1:
                    wait_s(1 - p)
                if m < nm - 1:
                    fire_g(m + 1, 1 - p)
                fire_s(m, p)
            wait_s((nm - 1) % 2)

        @pl.when(c == 0)
        def _():
            run(s * NM16_0, NM16_0)

        plsc.subcore_barrier()

        @pl.when(c == 0)
        def _():
            pltpu.sync_copy(acc_sh.at[pl.ds(base, RPT)],
                            acc_out.at[pl.ds(base, RPT)])

    return sc_pass


def _make_deg():
    scratch = [
        pltpu.VMEM((NMD_0, MEGA), jnp.int32),
        pltpu.VMEM((MEGA, 16), jnp.float32),
        pltpu.VMEM_SHARED((NP, 16), jnp.float32),
        pltpu.SemaphoreType.DMA,
    ]

    @functools.partial(
        pl.kernel, mesh=_MESH,
        out_type=(jax.ShapeDtypeStruct((NP, 16), jnp.float32),),
        scratch_types=scratch, compiler_params=_SC_PARAMS)
    def deg_pass(rows, deg_out, row_v, ones_v, deg_sh, dsem):
        c = lax.axis_index("c")
        s = lax.axis_index("s")
        base = s * RPT

        @pl.when(c == 0)
        def _():
            _zero_rows(ones_v, MEGA, 16)
            pltpu.sync_copy(ones_v.at[pl.ds(0, RPT)],
                            deg_sh.at[pl.ds(base, RPT)])
            _fill_ones(ones_v, MEGA, 16)
        plsc.subcore_barrier()

        def run(m0, nm):
            pltpu.sync_copy(rows.at[pl.ds(m0, nm)], row_v.at[pl.ds(0, nm)])
            for m in range(nm):
                pltpu.async_copy(ones_v, deg_sh.at[row_v.at[m]], dsem,
                                 add=True)
            for m in range(nm):
                pltpu.make_async_copy(ones_v, deg_sh.at[row_v.at[0]],
                                      dsem).wait()

        @pl.when(c == 0)
        def _():
            run(s * NMD_0, NMD_0)

        plsc.subcore_barrier()

        @pl.when(c == 0)
        def _():
            pltpu.sync_copy(deg_sh.at[pl.ds(base, RPT)],
                            deg_out.at[pl.ds(base, RPT)])

    return deg_pass


_sc_pass_128 = _make_sc_pass128()
_sc_pass_16 = _make_sc_pass16()
_sc_deg = _make_deg()


def _mu_y_body2(a, deg, x, mu_o, y_o):
    degc = jnp.maximum(deg[:, 0:1], 1.0)
    mu = a[...] / degc
    mu_o[...] = mu
    y_o[...] = (x[...] - mu) ** 2


def _sigma_h_body(sp, deg, x, wl, bl, wr, h_o):
    degc = jnp.maximum(deg[:, 0:1], 1.0)
    sig = sp[...] / degc
    sig = jnp.sqrt(jnp.where(sig == 0.0, 1e-16, sig))
    h = (jnp.dot(sig, wl[...], preferred_element_type=jnp.float32) + bl[...]
         + jnp.dot(x[...], wr[...], preferred_element_type=jnp.float32))
    h_o[...] = jnp.maximum(h, 0.0)


def _sigma_out_body(sp, deg, h, wl, bl, wr, o_o):
    degc = jnp.maximum(deg[:, 0:1], 1.0)
    sig = sp[...] / degc
    sig = jnp.sqrt(jnp.where(sig == 0.0, 1e-16, sig))
    o = (jnp.dot(sig, wl[...], preferred_element_type=jnp.float32) + bl[...]
         + jnp.dot(h[...], wr[...], preferred_element_type=jnp.float32))
    m = jnp.max(o, axis=1, keepdims=True)
    lse = jnp.log(jnp.sum(jnp.exp(o - m), axis=1, keepdims=True))
    o_o[...] = o - m - lse


def _rows_spec(w):
    return pl.BlockSpec((ROWS_BLK, w), lambda i: (i, 0))


def _full_spec(r, w):
    return pl.BlockSpec((r, w), lambda i: (0, 0))


_GRID = N // ROWS_BLK


def _mu_y(a, deg, x, w):
    return pl.pallas_call(
        _mu_y_body2,
        grid=(_GRID,),
        in_specs=[_rows_spec(w), _rows_spec(16), _rows_spec(w)],
        out_specs=[_rows_spec(w), _rows_spec(w)],
        out_shape=[jax.ShapeDtypeStruct((N, w), jnp.float32),
                   jax.ShapeDtypeStruct((N, w), jnp.float32)],
    )(a, deg, x)


def _sigma_h(sp, deg, x, wl, bl, wr):
    return pl.pallas_call(
        _sigma_h_body,
        grid=(_GRID,),
        in_specs=[_rows_spec(D), _rows_spec(16), _rows_spec(D),
                  _full_spec(D, H), _full_spec(1, H), _full_spec(D, H)],
        out_specs=_rows_spec(H),
        out_shape=jax.ShapeDtypeStruct((N, H), jnp.float32),
    )(sp, deg, x, wl, bl, wr)


def _sigma_out(sp, deg, h, wl, bl, wr):
    return pl.pallas_call(
        _sigma_out_body,
        grid=(_GRID,),
        in_specs=[_rows_spec(H), _rows_spec(16), _rows_spec(H),
                  _full_spec(H, C), _full_spec(1, C), _full_spec(H, C)],
        out_specs=_rows_spec(C),
        out_shape=jax.ShapeDtypeStruct((N, C), jnp.float32),
    )(sp, deg, h, wl, bl, wr)


def kernel(x, edge_index, Wl0, bl0, Wr0, Wl1, bl1, Wr1):
    row = edge_index[0]
    col = edge_index[1]
    pad = EPAD - E
    rowp = jnp.concatenate([row, jnp.full((pad,), N, jnp.int32)])
    colp = jnp.concatenate([col, jnp.zeros((pad,), jnp.int32)])
    rowm = rowp.reshape(NMM, MEGA)
    colm = colp.reshape(NMM, MEGA)
    rowp = rowp.reshape(NCH, CHUNK)
    colp = colp.reshape(NCH, CHUNK)

    bl0r = bl0.reshape(1, H)
    bl1r = bl1.reshape(1, C)

    deg_p, = _sc_deg(rowm)
    deg = deg_p[:N]
    mu_p, = _sc_pass_128(x, rowp, colp)
    mu, y = _mu_y(mu_p[:N], deg, x, D)
    sig_p, = _sc_pass_128(y, rowp, colp)
    h = _sigma_h(sig_p[:N], deg, x, Wl0, bl0r, Wr0)

    mu2_p, = _sc_pass_16(h, rowm, colm)
    mu2, y2 = _mu_y(mu2_p[:N], deg, h, H)
    sig2_p, = _sc_pass_16(y2, rowm, colm)
    out = _sigma_out(sig2_p[:N], deg, h, Wl1, bl1r, Wr1)
    return out


# speedup vs baseline: 1.3567x; 1.3567x over previous
import functools

import jax
import jax.numpy as jnp
from jax import lax
from jax.experimental import pallas as pl
from jax.experimental.pallas import tpu as pltpu
from jax.experimental.pallas import tpu_sc as plsc

N = 10000
E = 320000
D = 128
H = 16
C = 40

NC = 2
NS = 16
NW = NC * NS
CHUNK = 128
CPW = 80
IDX_BLK = 8
EPW = CPW * CHUNK
EPAD = NW * EPW
NP = 10112
RPT = NP // NS

ROWS_BLK = 1000

_MESH = plsc.VectorSubcoreMesh(core_axis_name="c", subcore_axis_name="s")
_SC_PARAMS = pltpu.CompilerParams(use_tc_tiling_on_sc=False)
MEGA = IDX_BLK * CHUNK

CH128_0 = 128
CH128_1 = 32
NM16_0 = 14
NM16_1 = 6
NMD_0 = 12
NMD_1 = 8
SPC = 32
NCH = EPAD // CHUNK
NMM = EPAD // MEGA


def _zero_rows(ref, nrows, w):
    def zr(i, carry):
        for k in range(w // 16):
            ref[i, pl.ds(k * 16, 16)] = jnp.zeros((16,), jnp.float32)
        return carry
    lax.fori_loop(0, nrows, zr, 0)


def _fill_ones(ref, nrows, w):
    def orow(i, carry):
        for k in range(w // 16):
            ref[i, pl.ds(k * 16, 16)] = jnp.ones((16,), jnp.float32)
        return carry
    lax.fori_loop(0, nrows, orow, 0)


def _make_sc_pass128():
    scratch = [
        pltpu.VMEM((SPC, CHUNK), jnp.int32),
        pltpu.VMEM((SPC, CHUNK), jnp.int32),
        pltpu.VMEM((2, CHUNK, D), jnp.float32),
        pltpu.VMEM_SHARED((NP, D), jnp.float32),
        pltpu.SemaphoreType.DMA,
        pltpu.SemaphoreType.DMA,
        pltpu.SemaphoreType.DMA,
        pltpu.SemaphoreType.DMA,
    ]

    @functools.partial(
        pl.kernel, mesh=_MESH,
        out_type=(jax.ShapeDtypeStruct((NC, NP, D), jnp.float32),),
        scratch_types=scratch, compiler_params=_SC_PARAMS)
    def sc_pass(feat, rows, cols, acc_out, row_v, col_v, g, acc_sh,
                gsem0, gsem1, ssem0, ssem1):
        c = lax.axis_index("c")
        s = lax.axis_index("s")
        base = s * RPT
        gsem = (gsem0, gsem1)
        ssem = (ssem0, ssem1)

        def fire_g(t, p):
            pltpu.async_copy(feat.at[col_v.at[t]], g.at[p], gsem[p])

        def wait_g(p):
            pltpu.make_async_copy(feat.at[col_v.at[0]], g.at[p], gsem[p]).wait()

        def fire_s(t, p):
            pltpu.async_copy(g.at[p], acc_sh.at[row_v.at[t]], ssem[p], add=True)

        def wait_s(p):
            pltpu.make_async_copy(g.at[p], acc_sh.at[row_v.at[0]], ssem[p]).wait()

        _zero_rows(g.at[0], CHUNK, D)
        for b in range(RPT // CHUNK):
            pltpu.sync_copy(g.at[0], acc_sh.at[pl.ds(base + b * CHUNK, CHUNK)])
        rem = RPT % CHUNK
        if rem:
            pltpu.sync_copy(g.at[0, pl.ds(0, rem)],
                            acc_sh.at[pl.ds(base + (RPT // CHUNK) * CHUNK, rem)])
        plsc.subcore_barrier()

        def run(chunk0, nstage):
            for stage in range(nstage):
                st = chunk0 + stage * SPC
                pltpu.sync_copy(rows.at[pl.ds(st, SPC)], row_v)
                pltpu.sync_copy(cols.at[pl.ds(st, SPC)], col_v)
                fire_g(0, 0)

                def body(jj2, carry):
                    tA = 2 * jj2
                    wait_g(0)

                    @pl.when(jj2 > 0)
                    def _():
                        wait_s(1)
                    fire_g(tA + 1, 1)
                    fire_s(tA, 0)
                    wait_g(1)
                    wait_s(0)

                    @pl.when(jj2 < SPC // 2 - 1)
                    def _():
                        fire_g(tA + 2, 0)
                    fire_s(tA + 1, 1)
                    return carry
                lax.fori_loop(0, SPC // 2, body, 0)
                wait_s(1)

        @pl.when(c == 0)
        def _():
            run(s * CH128_0, CH128_0 // SPC)

        @pl.when(c == 1)
        def _():
            run(NS * CH128_0 + s * CH128_1, CH128_1 // SPC)

        plsc.subcore_barrier()
        for b in range(RPT // CHUNK):
            sl = pl.ds(base + b * CHUNK, CHUNK)
            pltpu.sync_copy(acc_sh.at[sl], acc_out.at[c, sl])
        if rem:
            sl = pl.ds(base + (RPT // CHUNK) * CHUNK, rem)
            pltpu.sync_copy(acc_sh.at[sl], acc_out.at[c, sl])

    return sc_pass


def _make_sc_pass16():
    scratch = [
        pltpu.VMEM((NM16_0, MEGA), jnp.int32),
        pltpu.VMEM((NM16_0, MEGA), jnp.int32),
        pltpu.VMEM((2, MEGA, H), jnp.float32),
        pltpu.VMEM_SHARED((NP, H), jnp.float32),
        pltpu.SemaphoreType.DMA,
        pltpu.SemaphoreType.DMA,
        pltpu.SemaphoreType.DMA,
        pltpu.SemaphoreType.DMA,
    ]

    @functools.partial(
        pl.kernel, mesh=_MESH,
        out_type=(jax.ShapeDtypeStruct((NC, NP, H), jnp.float32),),
        scratch_types=scratch, compiler_params=_SC_PARAMS)
    def sc_pass(feat, rows, cols, acc_out, row_v, col_v, g, acc_sh,
                gsem0, gsem1, ssem0, ssem1):
        c = lax.axis_index("c")
        s = lax.axis_index("s")
        base = s * RPT
        gsem = (gsem0, gsem1)
        ssem = (ssem0, ssem1)

        def fire_g(m, p):
            pltpu.async_copy(feat.at[col_v.at[m]], g.at[p], gsem[p])

        def wait_g(p):
            pltpu.make_async_copy(feat.at[col_v.at[0]], g.at[p], gsem[p]).wait()

        def fire_s(m, p):
            pltpu.async_copy(g.at[p], acc_sh.at[row_v.at[m]], ssem[p], add=True)

        def wait_s(p):
            pltpu.make_async_copy(g.at[p], acc_sh.at[row_v.at[0]],
                                  ssem[p]).wait()

        _zero_rows(g.at[0], MEGA, H)
        pltpu.sync_copy(g.at[0, pl.ds(0, RPT)], acc_sh.at[pl.ds(base, RPT)])
        plsc.subcore_barrier()

        def run(m0, nm):
            pltpu.sync_copy(rows.at[pl.ds(m0, nm)], row_v.at[pl.ds(0, nm)])
            pltpu.sync_copy(cols.at[pl.ds(m0, nm)], col_v.at[pl.ds(0, nm)])
            fire_g(0, 0)
            for m in range(nm):
                p = m % 2
                wait_g(p)
                if m >= 1:
                    wait_s(1 - p)
                if m < nm - 1:
                    fire_g(m + 1, 1 - p)
                fire_s(m, p)
            wait_s((nm - 1) % 2)

        @pl.when(c == 0)
        def _():
            run(s * NM16_0, NM16_0)

        @pl.when(c == 1)
        def _():
            run(NS * NM16_0 + s * NM16_1, NM16_1)

        plsc.subcore_barrier()
        pltpu.sync_copy(acc_sh.at[pl.ds(base, RPT)],
                        acc_out.at[c, pl.ds(base, RPT)])

    return sc_pass


def _make_deg():
    scratch = [
        pltpu.VMEM((NMD_0, MEGA), jnp.int32),
        pltpu.VMEM((MEGA, 16), jnp.float32),
        pltpu.VMEM_SHARED((NP, 16), jnp.float32),
        pltpu.SemaphoreType.DMA,
    ]

    @functools.partial(
        pl.kernel, mesh=_MESH,
        out_type=(jax.ShapeDtypeStruct((NC, NP, 16), jnp.float32),),
        scratch_types=scratch, compiler_params=_SC_PARAMS)
    def deg_pass(rows, deg_out, row_v, ones_v, deg_sh, dsem):
        c = lax.axis_index("c")
        s = lax.axis_index("s")
        base = s * RPT

        _zero_rows(ones_v, MEGA, 16)
        pltpu.sync_copy(ones_v.at[pl.ds(0, RPT)], deg_sh.at[pl.ds(base, RPT)])
        _fill_ones(ones_v, MEGA, 16)
        plsc.subcore_barrier()

        def run(m0, nm):
            pltpu.sync_copy(rows.at[pl.ds(m0, nm)], row_v.at[pl.ds(0, nm)])
            for m in range(nm):
                pltpu.async_copy(ones_v, deg_sh.at[row_v.at[m]], dsem,
                                 add=True)
            for m in range(nm):
                pltpu.make_async_copy(ones_v, deg_sh.at[row_v.at[0]],
                                      dsem).wait()

        @pl.when(c == 0)
        def _():
            run(s * NMD_0, NMD_0)

        @pl.when(c == 1)
        def _():
            run(NS * NMD_0 + s * NMD_1, NMD_1)

        plsc.subcore_barrier()
        pltpu.sync_copy(deg_sh.at[pl.ds(base, RPT)],
                        deg_out.at[c, pl.ds(base, RPT)])

    return deg_pass


_sc_pass_128 = _make_sc_pass128()
_sc_pass_16 = _make_sc_pass16()
_sc_deg = _make_deg()


def _mu_y_deg_body(a0, a1, d0, d1, x, mu_o, y_o, deg_o):
    deg = d0[...] + d1[...]
    degc = jnp.maximum(deg[:, 0:1], 1.0)
    mu = (a0[...] + a1[...]) / degc
    mu_o[...] = mu
    y_o[...] = (x[...] - mu) ** 2
    deg_o[...] = deg


def _mu_y_body(a0, a1, deg, x, mu_o, y_o):
    degc = jnp.maximum(deg[:, 0:1], 1.0)
    mu = (a0[...] + a1[...]) / degc
    mu_o[...] = mu
    y_o[...] = (x[...] - mu) ** 2


def _sigma_h_body(s0, s1, deg, x, wl, bl, wr, h_o):
    degc = jnp.maximum(deg[:, 0:1], 1.0)
    sig = (s0[...] + s1[...]) / degc
    sig = jnp.sqrt(jnp.where(sig == 0.0, 1e-16, sig))
    h = (jnp.dot(sig, wl[...], preferred_element_type=jnp.float32) + bl[...]
         + jnp.dot(x[...], wr[...], preferred_element_type=jnp.float32))
    h_o[...] = jnp.maximum(h, 0.0)


def _sigma_out_body(s0, s1, deg, h, wl, bl, wr, o_o):
    degc = jnp.maximum(deg[:, 0:1], 1.0)
    sig = (s0[...] + s1[...]) / degc
    sig = jnp.sqrt(jnp.where(sig == 0.0, 1e-16, sig))
    o = (jnp.dot(sig, wl[...], preferred_element_type=jnp.float32) + bl[...]
         + jnp.dot(h[...], wr[...], preferred_element_type=jnp.float32))
    m = jnp.max(o, axis=1, keepdims=True)
    lse = jnp.log(jnp.sum(jnp.exp(o - m), axis=1, keepdims=True))
    o_o[...] = o - m - lse


def _rows_spec(w):
    return pl.BlockSpec((ROWS_BLK, w), lambda i: (i, 0))


def _full_spec(r, w):
    return pl.BlockSpec((r, w), lambda i: (0, 0))


_GRID = N // ROWS_BLK


def _mu_y_deg(a0, a1, d0, d1, x):
    return pl.pallas_call(
        _mu_y_deg_body,
        grid=(_GRID,),
        in_specs=[_rows_spec(D), _rows_spec(D), _rows_spec(16), _rows_spec(16),
                  _rows_spec(D)],
        out_specs=[_rows_spec(D), _rows_spec(D), _rows_spec(16)],
        out_shape=[jax.ShapeDtypeStruct((N, D), jnp.float32),
                   jax.ShapeDtypeStruct((N, D), jnp.float32),
                   jax.ShapeDtypeStruct((N, 16), jnp.float32)],
    )(a0, a1, d0, d1, x)


def _mu_y(a0, a1, deg, x, w):
    return pl.pallas_call(
        _mu_y_body,
        grid=(_GRID,),
        in_specs=[_rows_spec(w), _rows_spec(w), _rows_spec(16), _rows_spec(w)],
        out_specs=[_rows_spec(w), _rows_spec(w)],
        out_shape=[jax.ShapeDtypeStruct((N, w), jnp.float32),
                   jax.ShapeDtypeStruct((N, w), jnp.float32)],
    )(a0, a1, deg, x)


def _sigma_h(s0, s1, deg, x, wl, bl, wr):
    return pl.pallas_call(
        _sigma_h_body,
        grid=(_GRID,),
        in_specs=[_rows_spec(D), _rows_spec(D), _rows_spec(16), _rows_spec(D),
                  _full_spec(D, H), _full_spec(1, H), _full_spec(D, H)],
        out_specs=_rows_spec(H),
        out_shape=jax.ShapeDtypeStruct((N, H), jnp.float32),
    )(s0, s1, deg, x, wl, bl, wr)


def _sigma_out(s0, s1, deg, h, wl, bl, wr):
    return pl.pallas_call(
        _sigma_out_body,
        grid=(_GRID,),
        in_specs=[_rows_spec(H), _rows_spec(H), _rows_spec(16), _rows_spec(H),
                  _full_spec(H, C), _full_spec(1, C), _full_spec(H, C)],
        out_specs=_rows_spec(C),
        out_shape=jax.ShapeDtypeStruct((N, C), jnp.float32),
    )(s0, s1, deg, h, wl, bl, wr)


def kernel(x, edge_index, Wl0, bl0, Wr0, Wl1, bl1, Wr1):
    row = edge_index[0]
    col = edge_index[1]
    pad = EPAD - E
    rowp = jnp.concatenate([row, jnp.full((pad,), N, jnp.int32)])
    colp = jnp.concatenate([col, jnp.zeros((pad,), jnp.int32)])
    rowm = rowp.reshape(NMM, MEGA)
    colm = colp.reshape(NMM, MEGA)
    rowp = rowp.reshape(NCH, CHUNK)
    colp = colp.reshape(NCH, CHUNK)

    bl0r = bl0.reshape(1, H)
    bl1r = bl1.reshape(1, C)

    deg_p, = _sc_deg(rowm)
    mu_p, = _sc_pass_128(x, rowp, colp)
    mu, y, deg = _mu_y_deg(mu_p[0, :N], mu_p[1, :N],
                           deg_p[0, :N], deg_p[1, :N], x)
    sig_p, = _sc_pass_128(y, rowp, colp)
    h = _sigma_h(sig_p[0, :N], sig_p[1, :N], deg, x, Wl0, bl0r, Wr0)

    mu2_p, = _sc_pass_16(h, rowm, colm)
    mu2, y2 = _mu_y(mu2_p[0, :N], mu2_p[1, :N], deg, h, H)
    sig2_p, = _sc_pass_16(y2, rowm, colm)
    out = _sigma_out(sig2_p[0, :N], sig2_p[1, :N], deg, h, Wl1, bl1r, Wr1)
    return out


# speedup vs baseline: 1.3568x; 1.0000x over previous
import functools

import jax
import jax.numpy as jnp
from jax import lax
from jax.experimental import pallas as pl
from jax.experimental.pallas import tpu as pltpu
from jax.experimental.pallas import tpu_sc as plsc

N = 10000
E = 320000
D = 128
H = 16
C = 40

NC = 2
NS = 16
NW = NC * NS
CHUNK = 128
CPW = 80
IDX_BLK = 8
EPW = CPW * CHUNK
EPAD = NW * EPW
NP = 10112
RPT = NP // NS

ROWS_BLK = 1000

_MESH = plsc.VectorSubcoreMesh(core_axis_name="c", subcore_axis_name="s")
_SC_PARAMS = pltpu.CompilerParams(use_tc_tiling_on_sc=False)
MEGA = IDX_BLK * CHUNK

CH128_0 = 128
CH128_1 = 32
NM16_0 = 14
NM16_1 = 6
NMD_0 = 12
NMD_1 = 8
SPC = 32
NCH = EPAD // CHUNK
NMM = EPAD // MEGA


def _zero_rows(ref, nrows, w):
    def zr(i, carry):
        for k in range(w // 16):
            ref[i, pl.ds(k * 16, 16)] = jnp.zeros((16,), jnp.float32)
        return carry
    lax.fori_loop(0, nrows, zr, 0)


def _fill_ones(ref, nrows, w):
    def orow(i, carry):
        for k in range(w // 16):
            ref[i, pl.ds(k * 16, 16)] = jnp.ones((16,), jnp.float32)
        return carry
    lax.fori_loop(0, nrows, orow, 0)


def _make_sc_pass128():
    scratch = [
        pltpu.VMEM((SPC, CHUNK), jnp.int32),
        pltpu.VMEM((SPC, CHUNK), jnp.int32),
        pltpu.VMEM((2, CHUNK, D), jnp.float32),
        pltpu.VMEM_SHARED((NP, D), jnp.float32),
        pltpu.SemaphoreType.DMA,
        pltpu.SemaphoreType.DMA,
        pltpu.SemaphoreType.DMA,
        pltpu.SemaphoreType.DMA,
    ]

    @functools.partial(
        pl.kernel, mesh=_MESH,
        out_type=(jax.ShapeDtypeStruct((NC, NP, D), jnp.float32),),
        scratch_types=scratch, compiler_params=_SC_PARAMS)
    def sc_pass(feat, rows, cols, acc_out, row_v, col_v, g, acc_sh,
                gsem0, gsem1, ssem0, ssem1):
        c = lax.axis_index("c")
        s = lax.axis_index("s")
        base = s * RPT
        gsem = (gsem0, gsem1)
        ssem = (ssem0, ssem1)

        def fire_g(t, p):
            pltpu.async_copy(feat.at[col_v.at[t]], g.at[p], gsem[p])

        def wait_g(p):
            pltpu.make_async_copy(feat.at[col_v.at[0]], g.at[p], gsem[p]).wait()

        def fire_s(t, p):
            pltpu.async_copy(g.at[p], acc_sh.at[row_v.at[t]], ssem[p], add=True)

        def wait_s(p):
            pltpu.make_async_copy(g.at[p], acc_sh.at[row_v.at[0]], ssem[p]).wait()

        _zero_rows(g.at[0], CHUNK, D)
        nzb = (RPT + CHUNK - 1) // CHUNK
        def idxrow(i, carry):
            for k in range(CHUNK // 16):
                v = base + i * CHUNK + k * 16 + lax.iota(jnp.int32, 16)
                v = jnp.minimum(v, base + RPT - 1)
                row_v[i, pl.ds(k * 16, 16)] = v
            return carry
        lax.fori_loop(0, nzb, idxrow, 0)
        for b in range(nzb):
            pltpu.async_copy(g.at[0], acc_sh.at[row_v.at[b]], ssem0)
        for b in range(nzb):
            pltpu.make_async_copy(g.at[0], acc_sh.at[row_v.at[0]],
                                  ssem0).wait()
        rem = RPT % CHUNK
        plsc.subcore_barrier()

        def run(chunk0, nstage):
            for stage in range(nstage):
                st = chunk0 + stage * SPC
                pltpu.sync_copy(rows.at[pl.ds(st, SPC)], row_v)
                pltpu.sync_copy(cols.at[pl.ds(st, SPC)], col_v)
                fire_g(0, 0)

                def body(jj2, carry):
                    tA = 2 * jj2
                    wait_g(0)

                    @pl.when(jj2 > 0)
                    def _():
                        wait_s(1)
                    fire_g(tA + 1, 1)
                    fire_s(tA, 0)
                    wait_g(1)
                    wait_s(0)

                    @pl.when(jj2 < SPC // 2 - 1)
                    def _():
                        fire_g(tA + 2, 0)
                    fire_s(tA + 1, 1)
                    return carry
                lax.fori_loop(0, SPC // 2, body, 0)
                wait_s(1)

        @pl.when(c == 0)
        def _():
            run(s * CH128_0, CH128_0 // SPC)

        @pl.when(c == 1)
        def _():
            run(NS * CH128_0 + s * CH128_1, CH128_1 // SPC)

        plsc.subcore_barrier()
        for b in range(RPT // CHUNK):
            sl = pl.ds(base + b * CHUNK, CHUNK)
            pltpu.sync_copy(acc_sh.at[sl], acc_out.at[c, sl])
        if rem:
            sl = pl.ds(base + (RPT // CHUNK) * CHUNK, rem)
            pltpu.sync_copy(acc_sh.at[sl], acc_out.at[c, sl])

    return sc_pass


def _make_sc_pass16():
    scratch = [
        pltpu.VMEM((NM16_0, MEGA), jnp.int32),
        pltpu.VMEM((NM16_0, MEGA), jnp.int32),
        pltpu.VMEM((2, MEGA, H), jnp.float32),
        pltpu.VMEM_SHARED((NP, H), jnp.float32),
        pltpu.SemaphoreType.DMA,
        pltpu.SemaphoreType.DMA,
        pltpu.SemaphoreType.DMA,
        pltpu.SemaphoreType.DMA,
    ]

    @functools.partial(
        pl.kernel, mesh=_MESH,
        out_type=(jax.ShapeDtypeStruct((NC, NP, H), jnp.float32),),
        scratch_types=scratch, compiler_params=_SC_PARAMS)
    def sc_pass(feat, rows, cols, acc_out, row_v, col_v, g, acc_sh,
                gsem0, gsem1, ssem0, ssem1):
        c = lax.axis_index("c")
        s = lax.axis_index("s")
        base = s * RPT
        gsem = (gsem0, gsem1)
        ssem = (ssem0, ssem1)

        def fire_g(m, p):
            pltpu.async_copy(feat.at[col_v.at[m]], g.at[p], gsem[p])

        def wait_g(p):
            pltpu.make_async_copy(feat.at[col_v.at[0]], g.at[p], gsem[p]).wait()

        def fire_s(m, p):
            pltpu.async_copy(g.at[p], acc_sh.at[row_v.at[m]], ssem[p], add=True)

        def wait_s(p):
            pltpu.make_async_copy(g.at[p], acc_sh.at[row_v.at[0]],
                                  ssem[p]).wait()

        _zero_rows(g.at[0], MEGA, H)
        pltpu.sync_copy(g.at[0, pl.ds(0, RPT)], acc_sh.at[pl.ds(base, RPT)])
        plsc.subcore_barrier()

        def run(m0, nm):
            pltpu.sync_copy(rows.at[pl.ds(m0, nm)], row_v.at[pl.ds(0, nm)])
            pltpu.sync_copy(cols.at[pl.ds(m0, nm)], col_v.at[pl.ds(0, nm)])
            fire_g(0, 0)
            for m in range(nm):
                p = m % 2
                wait_g(p)
                if m >= 1:
                    wait_s(1 - p)
                if m < nm - 1:
                    fire_g(m + 1, 1 - p)
                fire_s(m, p)
            wait_s((nm - 1) % 2)

        @pl.when(c == 0)
        def _():
            run(s * NM16_0, NM16_0)

        @pl.when(c == 1)
        def _():
            run(NS * NM16_0 + s * NM16_1, NM16_1)

        plsc.subcore_barrier()
        pltpu.sync_copy(acc_sh.at[pl.ds(base, RPT)],
                        acc_out.at[c, pl.ds(base, RPT)])

    return sc_pass


def _make_deg():
    scratch = [
        pltpu.VMEM((NMD_0, MEGA), jnp.int32),
        pltpu.VMEM((MEGA, 16), jnp.float32),
        pltpu.VMEM_SHARED((NP, 16), jnp.float32),
        pltpu.SemaphoreType.DMA,
    ]

    @functools.partial(
        pl.kernel, mesh=_MESH,
        out_type=(jax.ShapeDtypeStruct((NC, NP, 16), jnp.float32),),
        scratch_types=scratch, compiler_params=_SC_PARAMS)
    def deg_pass(rows, deg_out, row_v, ones_v, deg_sh, dsem):
        c = lax.axis_index("c")
        s = lax.axis_index("s")
        base = s * RPT

        _zero_rows(ones_v, MEGA, 16)
        pltpu.sync_copy(ones_v.at[pl.ds(0, RPT)], deg_sh.at[pl.ds(base, RPT)])
        _fill_ones(ones_v, MEGA, 16)
        plsc.subcore_barrier()

        def run(m0, nm):
            pltpu.sync_copy(rows.at[pl.ds(m0, nm)], row_v.at[pl.ds(0, nm)])
            for m in range(nm):
                pltpu.async_copy(ones_v, deg_sh.at[row_v.at[m]], dsem,
                                 add=True)
            for m in range(nm):
                pltpu.make_async_copy(ones_v, deg_sh.at[row_v.at[0]],
                                      dsem).wait()

        @pl.when(c == 0)
        def _():
            run(s * NMD_0, NMD_0)

        @pl.when(c == 1)
        def _():
            run(NS * NMD_0 + s * NMD_1, NMD_1)

        plsc.subcore_barrier()
        pltpu.sync_copy(deg_sh.at[pl.ds(base, RPT)],
                        deg_out.at[c, pl.ds(base, RPT)])

    return deg_pass


_sc_pass_128 = _make_sc_pass128()
_sc_pass_16 = _make_sc_pass16()
_sc_deg = _make_deg()


def _mu_y_deg_body(a0, a1, d0, d1, x, mu_o, y_o, deg_o):
    deg = d0[...] + d1[...]
    degc = jnp.maximum(deg[:, 0:1], 1.0)
    mu = (a0[...] + a1[...]) / degc
    mu_o[...] = mu
    y_o[...] = (x[...] - mu) ** 2
    deg_o[...] = deg


def _mu_y_body(a0, a1, deg, x, mu_o, y_o):
    degc = jnp.maximum(deg[:, 0:1], 1.0)
    mu = (a0[...] + a1[...]) / degc
    mu_o[...] = mu
    y_o[...] = (x[...] - mu) ** 2


def _sigma_h_body(s0, s1, deg, x, wl, bl, wr, h_o):
    degc = jnp.maximum(deg[:, 0:1], 1.0)
    sig = (s0[...] + s1[...]) / degc
    sig = jnp.sqrt(jnp.where(sig == 0.0, 1e-16, sig))
    h = (jnp.dot(sig, wl[...], preferred_element_type=jnp.float32) + bl[...]
         + jnp.dot(x[...], wr[...], preferred_element_type=jnp.float32))
    h_o[...] = jnp.maximum(h, 0.0)


def _sigma_out_body(s0, s1, deg, h, wl, bl, wr, o_o):
    degc = jnp.maximum(deg[:, 0:1], 1.0)
    sig = (s0[...] + s1[...]) / degc
    sig = jnp.sqrt(jnp.where(sig == 0.0, 1e-16, sig))
    o = (jnp.dot(sig, wl[...], preferred_element_type=jnp.float32) + bl[...]
         + jnp.dot(h[...], wr[...], preferred_element_type=jnp.float32))
    m = jnp.max(o, axis=1, keepdims=True)
    lse = jnp.log(jnp.sum(jnp.exp(o - m), axis=1, keepdims=True))
    o_o[...] = o - m - lse


def _rows_spec(w):
    return pl.BlockSpec((ROWS_BLK, w), lambda i: (i, 0))


def _full_spec(r, w):
    return pl.BlockSpec((r, w), lambda i: (0, 0))


_GRID = N // ROWS_BLK


def _mu_y_deg(a0, a1, d0, d1, x):
    return pl.pallas_call(
        _mu_y_deg_body,
        grid=(_GRID,),
        in_specs=[_rows_spec(D), _rows_spec(D), _rows_spec(16), _rows_spec(16),
                  _rows_spec(D)],
        out_specs=[_rows_spec(D), _rows_spec(D), _rows_spec(16)],
        out_shape=[jax.ShapeDtypeStruct((N, D), jnp.float32),
                   jax.ShapeDtypeStruct((N, D), jnp.float32),
                   jax.ShapeDtypeStruct((N, 16), jnp.float32)],
    )(a0, a1, d0, d1, x)


def _mu_y(a0, a1, deg, x, w):
    return pl.pallas_call(
        _mu_y_body,
        grid=(_GRID,),
        in_specs=[_rows_spec(w), _rows_spec(w), _rows_spec(16), _rows_spec(w)],
        out_specs=[_rows_spec(w), _rows_spec(w)],
        out_shape=[jax.ShapeDtypeStruct((N, w), jnp.float32),
                   jax.ShapeDtypeStruct((N, w), jnp.float32)],
    )(a0, a1, deg, x)


def _sigma_h(s0, s1, deg, x, wl, bl, wr):
    return pl.pallas_call(
        _sigma_h_body,
        grid=(_GRID,),
        in_specs=[_rows_spec(D), _rows_spec(D), _rows_spec(16), _rows_spec(D),
                  _full_spec(D, H), _full_spec(1, H), _full_spec(D, H)],
        out_specs=_rows_spec(H),
        out_shape=jax.ShapeDtypeStruct((N, H), jnp.float32),
    )(s0, s1, deg, x, wl, bl, wr)


def _sigma_out(s0, s1, deg, h, wl, bl, wr):
    return pl.pallas_call(
        _sigma_out_body,
        grid=(_GRID,),
        in_specs=[_rows_spec(H), _rows_spec(H), _rows_spec(16), _rows_spec(H),
                  _full_spec(H, C), _full_spec(1, C), _full_spec(H, C)],
        out_specs=_rows_spec(C),
        out_shape=jax.ShapeDtypeStruct((N, C), jnp.float32),
    )(s0, s1, deg, h, wl, bl, wr)


def kernel(x, edge_index, Wl0, bl0, Wr0, Wl1, bl1, Wr1):
    row = edge_index[0]
    col = edge_index[1]
    pad = EPAD - E
    rowp = jnp.concatenate([row, jnp.full((pad,), N, jnp.int32)])
    colp = jnp.concatenate([col, jnp.zeros((pad,), jnp.int32)])
    rowm = rowp.reshape(NMM, MEGA)
    colm = colp.reshape(NMM, MEGA)
    rowp = rowp.reshape(NCH, CHUNK)
    colp = colp.reshape(NCH, CHUNK)

    bl0r = bl0.reshape(1, H)
    bl1r = bl1.reshape(1, C)

    deg_p, = _sc_deg(rowm)
    mu_p, = _sc_pass_128(x, rowp, colp)
    mu, y, deg = _mu_y_deg(mu_p[0, :N], mu_p[1, :N],
                           deg_p[0, :N], deg_p[1, :N], x)
    sig_p, = _sc_pass_128(y, rowp, colp)
    h = _sigma_h(sig_p[0, :N], sig_p[1, :N], deg, x, Wl0, bl0r, Wr0)

    mu2_p, = _sc_pass_16(h, rowm, colm)
    mu2, y2 = _mu_y(mu2_p[0, :N], mu2_p[1, :N], deg, h, H)
    sig2_p, = _sc_pass_16(y2, rowm, colm)
    out = _sigma_out(sig2_p[0, :N], sig2_p[1, :N], deg, h, Wl1, bl1r, Wr1)
    return out


# speedup vs baseline: 1.3577x; 1.0007x over previous
import functools

import jax
import jax.numpy as jnp
from jax import lax
from jax.experimental import pallas as pl
from jax.experimental.pallas import tpu as pltpu
from jax.experimental.pallas import tpu_sc as plsc

N = 10000
E = 320000
D = 128
H = 16
C = 40

NC = 2
NS = 16
NW = NC * NS
CHUNK = 128
CPW = 80
IDX_BLK = 8
EPW = CPW * CHUNK
EPAD = NW * EPW
NP = 10112
RPT = NP // NS

ROWS_BLK = 1000

_MESH = plsc.VectorSubcoreMesh(core_axis_name="c", subcore_axis_name="s")
_SC_PARAMS = pltpu.CompilerParams(use_tc_tiling_on_sc=False)
MEGA = IDX_BLK * CHUNK

CH128_0 = 128
CH128_1 = 32
NM16_0 = 14
NM16_1 = 6
NMD_0 = 12
NMD_1 = 8
SPC = 32
NCH = EPAD // CHUNK
NMM = EPAD // MEGA


def _zero_rows(ref, nrows, w):
    def zr(i, carry):
        for k in range(w // 16):
            ref[i, pl.ds(k * 16, 16)] = jnp.zeros((16,), jnp.float32)
        return carry
    lax.fori_loop(0, nrows, zr, 0)


def _fill_ones(ref, nrows, w):
    def orow(i, carry):
        for k in range(w // 16):
            ref[i, pl.ds(k * 16, 16)] = jnp.ones((16,), jnp.float32)
        return carry
    lax.fori_loop(0, nrows, orow, 0)


def _make_sc_pass128():
    scratch = [
        pltpu.VMEM((SPC, CHUNK), jnp.int32),
        pltpu.VMEM((SPC, CHUNK), jnp.int32),
        pltpu.VMEM((2, CHUNK, D), jnp.float32),
        pltpu.VMEM_SHARED((NP, D), jnp.float32),
        pltpu.SemaphoreType.DMA,
        pltpu.SemaphoreType.DMA,
        pltpu.SemaphoreType.DMA,
        pltpu.SemaphoreType.DMA,
    ]

    @functools.partial(
        pl.kernel, mesh=_MESH,
        out_type=(jax.ShapeDtypeStruct((NC, NP, D), jnp.float32),),
        scratch_types=scratch, compiler_params=_SC_PARAMS)
    def sc_pass(feat, rows, cols, acc_out, row_v, col_v, g, acc_sh,
                gsem0, gsem1, ssem0, ssem1):
        c = lax.axis_index("c")
        s = lax.axis_index("s")
        base = s * RPT
        gsem = (gsem0, gsem1)
        ssem = (ssem0, ssem1)

        def fire_g(t, p):
            pltpu.async_copy(feat.at[col_v.at[t]], g.at[p], gsem[p])

        def wait_g(p):
            pltpu.make_async_copy(feat.at[col_v.at[0]], g.at[p], gsem[p]).wait()

        def fire_s(t, p):
            pltpu.async_copy(g.at[p], acc_sh.at[row_v.at[t]], ssem[p], add=True)

        def wait_s(p):
            pltpu.make_async_copy(g.at[p], acc_sh.at[row_v.at[0]], ssem[p]).wait()

        _zero_rows(g.at[0], CHUNK, D)
        for b in range(RPT // CHUNK):
            pltpu.sync_copy(g.at[0], acc_sh.at[pl.ds(base + b * CHUNK, CHUNK)])
        rem = RPT % CHUNK
        if rem:
            pltpu.sync_copy(g.at[0, pl.ds(0, rem)],
                            acc_sh.at[pl.ds(base + (RPT // CHUNK) * CHUNK, rem)])
        plsc.subcore_barrier()

        def run(chunk0, nstage):
            for stage in range(nstage):
                st = chunk0 + stage * SPC
                pltpu.sync_copy(rows.at[pl.ds(st, SPC)], row_v)
                pltpu.sync_copy(cols.at[pl.ds(st, SPC)], col_v)
                fire_g(0, 0)

                def body(jj2, carry):
                    tA = 2 * jj2
                    wait_g(0)

                    @pl.when(jj2 > 0)
                    def _():
                        wait_s(1)
                    fire_g(tA + 1, 1)
                    fire_s(tA, 0)
                    wait_g(1)
                    wait_s(0)

                    @pl.when(jj2 < SPC // 2 - 1)
                    def _():
                        fire_g(tA + 2, 0)
                    fire_s(tA + 1, 1)
                    return carry
                lax.fori_loop(0, SPC // 2, body, 0)
                wait_s(1)

        @pl.when(c == 0)
        def _():
            run(s * CH128_0, CH128_0 // SPC)

        @pl.when(c == 1)
        def _():
            run(NS * CH128_0 + s * CH128_1, CH128_1 // SPC)

        plsc.subcore_barrier()
        for b in range(RPT // CHUNK):
            sl = pl.ds(base + b * CHUNK, CHUNK)
            pltpu.sync_copy(acc_sh.at[sl], acc_out.at[c, sl])
        if rem:
            sl = pl.ds(base + (RPT // CHUNK) * CHUNK, rem)
            pltpu.sync_copy(acc_sh.at[sl], acc_out.at[c, sl])

    return sc_pass


def _make_sc_pass16():
    scratch = [
        pltpu.VMEM((NM16_0, MEGA), jnp.int32),
        pltpu.VMEM((NM16_0, MEGA), jnp.int32),
        pltpu.VMEM((2, MEGA, H), jnp.float32),
        pltpu.VMEM_SHARED((NP, H), jnp.float32),
        pltpu.SemaphoreType.DMA,
        pltpu.SemaphoreType.DMA,
        pltpu.SemaphoreType.DMA,
        pltpu.SemaphoreType.DMA,
    ]

    @functools.partial(
        pl.kernel, mesh=_MESH,
        out_type=(jax.ShapeDtypeStruct((NC, NP, H), jnp.float32),),
        scratch_types=scratch, compiler_params=_SC_PARAMS)
    def sc_pass(feat, rows, cols, acc_out, row_v, col_v, g, acc_sh,
                gsem0, gsem1, ssem0, ssem1):
        c = lax.axis_index("c")
        s = lax.axis_index("s")
        base = s * RPT
        gsem = (gsem0, gsem1)
        ssem = (ssem0, ssem1)

        def fire_g(m, p):
            pltpu.async_copy(feat.at[col_v.at[m]], g.at[p], gsem[p])

        def wait_g(p):
            pltpu.make_async_copy(feat.at[col_v.at[0]], g.at[p], gsem[p]).wait()

        def fire_s(m, p):
            pltpu.async_copy(g.at[p], acc_sh.at[row_v.at[m]], ssem[p], add=True)

        def wait_s(p):
            pltpu.make_async_copy(g.at[p], acc_sh.at[row_v.at[0]],
                                  ssem[p]).wait()

        _zero_rows(g.at[0], MEGA, H)
        pltpu.sync_copy(g.at[0, pl.ds(0, RPT)], acc_sh.at[pl.ds(base, RPT)])
        plsc.subcore_barrier()

        def run(m0, nm):
            pltpu.sync_copy(rows.at[pl.ds(m0, nm)], row_v.at[pl.ds(0, nm)])
            pltpu.sync_copy(cols.at[pl.ds(m0, nm)], col_v.at[pl.ds(0, nm)])
            fire_g(0, 0)
            for m in range(nm):
                p = m % 2
                wait_g(p)
                if m >= 1:
                    wait_s(1 - p)
                if m < nm - 1:
                    fire_g(m + 1, 1 - p)
                fire_s(m, p)
            wait_s((nm - 1) % 2)

        @pl.when(c == 0)
        def _():
            run(s * NM16_0, NM16_0)

        @pl.when(c == 1)
        def _():
            run(NS * NM16_0 + s * NM16_1, NM16_1)

        plsc.subcore_barrier()
        pltpu.sync_copy(acc_sh.at[pl.ds(base, RPT)],
                        acc_out.at[c, pl.ds(base, RPT)])

    return sc_pass


def _make_deg():
    scratch = [
        pltpu.VMEM((NMD_0, MEGA), jnp.int32),
        pltpu.VMEM((MEGA, 16), jnp.float32),
        pltpu.VMEM_SHARED((NP, 16), jnp.float32),
        pltpu.SemaphoreType.DMA,
    ]

    @functools.partial(
        pl.kernel, mesh=_MESH,
        out_type=(jax.ShapeDtypeStruct((NC, NP, 16), jnp.float32),),
        scratch_types=scratch, compiler_params=_SC_PARAMS)
    def deg_pass(rows, deg_out, row_v, ones_v, deg_sh, dsem):
        c = lax.axis_index("c")
        s = lax.axis_index("s")
        base = s * RPT

        _zero_rows(ones_v, MEGA, 16)
        pltpu.sync_copy(ones_v.at[pl.ds(0, RPT)], deg_sh.at[pl.ds(base, RPT)])
        _fill_ones(ones_v, MEGA, 16)
        plsc.subcore_barrier()

        def run(m0, nm):
            pltpu.sync_copy(rows.at[pl.ds(m0, nm)], row_v.at[pl.ds(0, nm)])
            for m in range(nm):
                pltpu.async_copy(ones_v, deg_sh.at[row_v.at[m]], dsem,
                                 add=True)
            for m in range(nm):
                pltpu.make_async_copy(ones_v, deg_sh.at[row_v.at[0]],
                                      dsem).wait()

        @pl.when(c == 0)
        def _():
            run(s * NMD_0, NMD_0)

        @pl.when(c == 1)
        def _():
            run(NS * NMD_0 + s * NMD_1, NMD_1)

        plsc.subcore_barrier()
        pltpu.sync_copy(deg_sh.at[pl.ds(base, RPT)],
                        deg_out.at[c, pl.ds(base, RPT)])

    return deg_pass


_sc_pass_128 = _make_sc_pass128()
_sc_pass_16 = _make_sc_pass16()
_sc_deg = _make_deg()


def _mu_y_deg_body(a0, a1, d0, d1, x, mu_o, y_o, deg_o):
    deg = d0[...] + d1[...]
    degc = jnp.maximum(deg[:, 0:1], 1.0)
    mu = (a0[...] + a1[...]) / degc
    mu_o[...] = mu
    y_o[...] = (x[...] - mu) ** 2
    deg_o[...] = deg


def _mu_y_body(a0, a1, deg, x, mu_o, y_o):
    degc = jnp.maximum(deg[:, 0:1], 1.0)
    mu = (a0[...] + a1[...]) / degc
    mu_o[...] = mu
    y_o[...] = (x[...] - mu) ** 2


def _sigma_h_body(s0, s1, deg, x, wl, bl, wr, h_o):
    degc = jnp.maximum(deg[:, 0:1], 1.0)
    sig = (s0[...] + s1[...]) / degc
    sig = jnp.sqrt(jnp.where(sig == 0.0, 1e-16, sig))
    h = (jnp.dot(sig, wl[...], preferred_element_type=jnp.float32) + bl[...]
         + jnp.dot(x[...], wr[...], preferred_element_type=jnp.float32))
    h_o[...] = jnp.maximum(h, 0.0)


def _sigma_out_body(s0, s1, deg, h, wl, bl, wr, o_o):
    degc = jnp.maximum(deg[:, 0:1], 1.0)
    sig = (s0[...] + s1[...]) / degc
    sig = jnp.sqrt(jnp.where(sig == 0.0, 1e-16, sig))
    o = (jnp.dot(sig, wl[...], preferred_element_type=jnp.float32) + bl[...]
         + jnp.dot(h[...], wr[...], preferred_element_type=jnp.float32))
    m = jnp.max(o, axis=1, keepdims=True)
    lse = jnp.log(jnp.sum(jnp.exp(o - m), axis=1, keepdims=True))
    o_o[...] = o - m - lse


def _rows_spec(w):
    return pl.BlockSpec((ROWS_BLK, w), lambda i: (i, 0))


def _full_spec(r, w):
    return pl.BlockSpec((r, w), lambda i: (0, 0))


_GRID = N // ROWS_BLK


def _mu_y_deg(a0, a1, d0, d1, x):
    return pl.pallas_call(
        _mu_y_deg_body,
        grid=(_GRID,),
        in_specs=[_rows_spec(D), _rows_spec(D), _rows_spec(16), _rows_spec(16),
                  _rows_spec(D)],
        out_specs=[_rows_spec(D), _rows_spec(D), _rows_spec(16)],
        out_shape=[jax.ShapeDtypeStruct((N, D), jnp.float32),
                   jax.ShapeDtypeStruct((N, D), jnp.float32),
                   jax.ShapeDtypeStruct((N, 16), jnp.float32)],
    )(a0, a1, d0, d1, x)


def _mu_y(a0, a1, deg, x, w):
    return pl.pallas_call(
        _mu_y_body,
        grid=(_GRID,),
        in_specs=[_rows_spec(w), _rows_spec(w), _rows_spec(16), _rows_spec(w)],
        out_specs=[_rows_spec(w), _rows_spec(w)],
        out_shape=[jax.ShapeDtypeStruct((N, w), jnp.float32),
                   jax.ShapeDtypeStruct((N, w), jnp.float32)],
    )(a0, a1, deg, x)


def _sigma_h(s0, s1, deg, x, wl, bl, wr):
    return pl.pallas_call(
        _sigma_h_body,
        grid=(_GRID,),
        in_specs=[_rows_spec(D), _rows_spec(D), _rows_spec(16), _rows_spec(D),
                  _full_spec(D, H), _full_spec(1, H), _full_spec(D, H)],
        out_specs=_rows_spec(H),
        out_shape=jax.ShapeDtypeStruct((N, H), jnp.float32),
    )(s0, s1, deg, x, wl, bl, wr)


def _sigma_out(s0, s1, deg, h, wl, bl, wr):
    return pl.pallas_call(
        _sigma_out_body,
        grid=(_GRID,),
        in_specs=[_rows_spec(H), _rows_spec(H), _rows_spec(16), _rows_spec(H),
                  _full_spec(H, C), _full_spec(1, C), _full_spec(H, C)],
        out_specs=_rows_spec(C),
        out_shape=jax.ShapeDtypeStruct((N, C), jnp.float32),
    )(s0, s1, deg, h, wl, bl, wr)


def kernel(x, edge_index, Wl0, bl0, Wr0, Wl1, bl1, Wr1):
    row = edge_index[0]
    col = edge_index[1]
    pad = EPAD - E
    rowp = jnp.concatenate([row, jnp.full((pad,), N, jnp.int32)])
    colp = jnp.concatenate([col, jnp.zeros((pad,), jnp.int32)])
    rowm = rowp.reshape(NMM, MEGA)
    colm = colp.reshape(NMM, MEGA)
    rowp = rowp.reshape(NCH, CHUNK)
    colp = colp.reshape(NCH, CHUNK)

    bl0r = bl0.reshape(1, H)
    bl1r = bl1.reshape(1, C)

    deg_p, = _sc_deg(rowm)
    mu_p, = _sc_pass_128(x, rowp, colp)
    mu, y, deg = _mu_y_deg(mu_p[0, :N], mu_p[1, :N],
                           deg_p[0, :N], deg_p[1, :N], x)
    sig_p, = _sc_pass_128(y, rowp, colp)
    h = _sigma_h(sig_p[0, :N], sig_p[1, :N], deg, x, Wl0, bl0r, Wr0)

    mu2_p, = _sc_pass_16(h, rowm, colm)
    mu2, y2 = _mu_y(mu2_p[0, :N], mu2_p[1, :N], deg, h, H)
    sig2_p, = _sc_pass_16(y2, rowm, colm)
    out = _sigma_out(sig2_p[0, :N], sig2_p[1, :N], deg, h, Wl1, bl1r, Wr1)
    return out
